# baseline (speedup 1.0000x reference)
import functools

import jax
import jax.numpy as jnp
from jax import lax
from jax.experimental import pallas as pl
from jax.experimental.pallas import tpu as pltpu
from jax.experimental.pallas import tpu_sc as plsc

_L = 16

_mesh = plsc.VectorSubcoreMesh(core_axis_name="c", subcore_axis_name="s")


def _take16(vec, idx):
    dnums = lax.GatherDimensionNumbers(
        offset_dims=(), collapsed_slice_dims=(0,), start_index_map=(0,))
    return lax.gather(
        vec, idx[:, None], dnums, (1,),
        mode=lax.GatherScatterMode.PROMISE_IN_BOUNDS)


@functools.partial(
    pl.kernel,
    mesh=_mesh,
    out_type=jax.ShapeDtypeStruct((13,), jnp.float32),
    scratch_types=[
        pltpu.VMEM((3 * _L,), jnp.float32),
        pltpu.SemaphoreType.DMA,
    ],
)
def _assign_sc(y_hbm, w_hbm, c_hbm, out_hbm, buf, sem):
    cid = lax.axis_index("c")
    sid = lax.axis_index("s")

    @pl.when(jnp.logical_and(cid == 0, sid == 0))
    def _():
        cp_y = pltpu.async_copy(y_hbm, buf.at[pl.ds(0, 13)], sem)
        cp_c = pltpu.async_copy(c_hbm, buf.at[pl.ds(_L, 13)], sem)
        cp_w = pltpu.async_copy(w_hbm, buf.at[pl.ds(2 * _L, 13)], sem)
        cp_y.wait()
        cp_c.wait()
        cp_w.wait()

        lane = lax.iota(jnp.int32, _L)
        yy = buf[pl.ds(0, _L)]
        cc = buf[pl.ds(_L, _L)]
        ww = buf[pl.ds(2 * _L, _L)]
        i0 = jnp.int32(0)
        idx_c = jnp.where(
            lane == 0, jnp.int32(1),
            jnp.where(lane == 2, jnp.int32(2), i0))
        idx_ay = jnp.where(lane == 1, jnp.int32(3), i0)
        idx_b = jnp.where(
            lane == 0, jnp.int32(2),
            jnp.where(lane == 1, jnp.int32(4),
                      jnp.where(lane == 3, jnp.int32(1), i0)))
        ta = _take16(cc, idx_c)
        tb = _take16(yy, idx_ay)
        ga = jnp.where(lane == 1, tb, ta)
        gb = _take16(yy, idx_b)
        sign = jnp.where(lane == 1, jnp.float32(1.0), jnp.float32(-1.0))
        val = ga + sign * gb
        res = jnp.where(lane < 4, val, ww)
        buf[pl.ds(2 * _L, _L)] = res
        pltpu.sync_copy(buf.at[pl.ds(2 * _L, 13)], out_hbm)


def kernel(y, w, c, t):
    return _assign_sc(y, w, c)


# speedup vs baseline: 1.0722x; 1.0722x over previous
import functools

import jax
import jax.numpy as jnp
from jax import lax
from jax.experimental import pallas as pl
from jax.experimental.pallas import tpu as pltpu
from jax.experimental.pallas import tpu_sc as plsc

_L = 16

_mesh = plsc.VectorSubcoreMesh(
    core_axis_name="c", subcore_axis_name="s", num_cores=1, num_subcores=1)


def _take16(vec, idx):
    dnums = lax.GatherDimensionNumbers(
        offset_dims=(), collapsed_slice_dims=(0,), start_index_map=(0,))
    return lax.gather(
        vec, idx[:, None], dnums, (1,),
        mode=lax.GatherScatterMode.PROMISE_IN_BOUNDS)


@functools.partial(
    pl.kernel,
    mesh=_mesh,
    out_type=jax.ShapeDtypeStruct((13,), jnp.float32),
    scratch_types=[
        pltpu.VMEM((3 * _L,), jnp.float32),
        pltpu.SemaphoreType.DMA,
    ],
)
def _assign_sc(y_hbm, w_hbm, c_hbm, out_hbm, buf, sem):
    cid = lax.axis_index("c")
    sid = lax.axis_index("s")

    @pl.when(jnp.logical_and(cid == 0, sid == 0))
    def _():
        cp_y = pltpu.async_copy(y_hbm, buf.at[pl.ds(0, 13)], sem)
        cp_c = pltpu.async_copy(c_hbm, buf.at[pl.ds(_L, 13)], sem)
        cp_w = pltpu.async_copy(w_hbm, buf.at[pl.ds(2 * _L, 13)], sem)
        cp_y.wait()
        cp_c.wait()
        cp_w.wait()

        lane = lax.iota(jnp.int32, _L)
        yy = buf[pl.ds(0, _L)]
        cc = buf[pl.ds(_L, _L)]
        ww = buf[pl.ds(2 * _L, _L)]
        i0 = jnp.int32(0)
        idx_c = jnp.where(
            lane == 0, jnp.int32(1),
            jnp.where(lane == 2, jnp.int32(2), i0))
        idx_ay = jnp.where(lane == 1, jnp.int32(3), i0)
        idx_b = jnp.where(
            lane == 0, jnp.int32(2),
            jnp.where(lane == 1, jnp.int32(4),
                      jnp.where(lane == 3, jnp.int32(1), i0)))
        ta = _take16(cc, idx_c)
        tb = _take16(yy, idx_ay)
        ga = jnp.where(lane == 1, tb, ta)
        gb = _take16(yy, idx_b)
        sign = jnp.where(lane == 1, jnp.float32(1.0), jnp.float32(-1.0))
        val = ga + sign * gb
        res = jnp.where(lane < 4, val, ww)
        buf[pl.ds(2 * _L, _L)] = res
        pltpu.sync_copy(buf.at[pl.ds(2 * _L, 13)], out_hbm)


def kernel(y, w, c, t):
    return _assign_sc(y, w, c)


# speedup vs baseline: 1.1561x; 1.0783x over previous
import functools

import jax
import jax.numpy as jnp
from jax.experimental import pallas as pl
from jax.experimental.pallas import tpu as pltpu
from jax.experimental.pallas import tpu_sc as plsc

_smesh = plsc.ScalarSubcoreMesh(axis_name="c", num_cores=1)


@functools.partial(
    pl.kernel,
    mesh=_smesh,
    out_type=jax.ShapeDtypeStruct((13,), jnp.float32),
    scratch_types=[
        pltpu.SMEM((13,), jnp.float32),
        pltpu.SMEM((13,), jnp.float32),
        pltpu.SMEM((13,), jnp.float32),
        pltpu.SemaphoreType.DMA,
    ],
)
def _assign_scs(y_hbm, w_hbm, c_hbm, out_hbm, ys, cs, ws, sem):
    cp_y = pltpu.async_copy(y_hbm, ys, sem)
    cp_c = pltpu.async_copy(c_hbm, cs, sem)
    cp_w = pltpu.async_copy(w_hbm, ws, sem)
    cp_y.wait()
    cp_c.wait()
    cp_w.wait()
    ws[0] = cs[1] - ys[2]
    ws[1] = ys[3] + ys[4]
    ws[2] = cs[2] - ys[0]
    ws[3] = cs[0] - ys[1]
    pltpu.sync_copy(ws, out_hbm)


def kernel(y, w, c, t):
    return _assign_scs(y, w, c)


# speedup vs baseline: 1.1602x; 1.0035x over previous
import functools

import jax
import jax.numpy as jnp
from jax.experimental import pallas as pl
from jax.experimental.pallas import tpu as pltpu
from jax.experimental.pallas import tpu_sc as plsc

_smesh = plsc.ScalarSubcoreMesh(axis_name="c", num_cores=1)


@functools.partial(
    pl.kernel,
    mesh=_smesh,
    out_type=jax.ShapeDtypeStruct((13,), jnp.float32),
    compiler_params=pltpu.CompilerParams(skip_device_barrier=True),
    scratch_types=[
        pltpu.SMEM((13,), jnp.float32),
        pltpu.SMEM((13,), jnp.float32),
        pltpu.SMEM((13,), jnp.float32),
        pltpu.SemaphoreType.DMA,
    ],
)
def _assign_scs(y_hbm, w_hbm, c_hbm, out_hbm, ys, cs, ws, sem):
    cp_y = pltpu.async_copy(y_hbm, ys, sem)
    cp_c = pltpu.async_copy(c_hbm, cs, sem)
    cp_w = pltpu.async_copy(w_hbm, ws, sem)
    cp_y.wait()
    cp_c.wait()
    cp_w.wait()
    ws[0] = cs[1] - ys[2]
    ws[1] = ys[3] + ys[4]
    ws[2] = cs[2] - ys[0]
    ws[3] = cs[0] - ys[1]
    pltpu.sync_copy(ws, out_hbm)


def kernel(y, w, c, t):
    return _assign_scs(y, w, c)
